# baseline (speedup 1.0000x reference)
import dataclasses
import functools

import jax
import jax.numpy as jnp
from jax import lax
from jax.experimental import pallas as pl
from jax.experimental.pallas import tpu as pltpu
from jax.experimental.pallas import tpu_sc as plsc

N = 10000
D = 128
E = 320000

NP = 10240
NH = NP // 2
NHD = 5248
CHUNK = 128
NW = 32
CPW = 80
EP = NW * CPW * CHUNK
RS = NP // 16
CAP = CPW * CHUNK

_mesh = plsc.VectorSubcoreMesh(core_axis_name="c", subcore_axis_name="s")

_cp = pltpu.CompilerParams()
if "needs_layout_passes" in pltpu.CompilerParams.__dataclass_fields__:
    _cp = dataclasses.replace(_cp, needs_layout_passes=False)


@functools.partial(
    pl.kernel,
    out_type=jax.ShapeDtypeStruct((2, NP, D), jnp.float32),
    mesh=_mesh,
    scratch_types=[
        pltpu.VMEM((CPW, CHUNK), jnp.int32),
        pltpu.VMEM((CHUNK, D), jnp.float32),
        pltpu.VMEM_SHARED((NP, D), jnp.float32),
    ],
)
def _hist_kernel(col_hbm, ones_hbm, zeros_hbm, out_hbm, col_v, ones_v, acc_sh):
    c = lax.axis_index("c")
    s = lax.axis_index("s")
    w = s * 2 + c
    pltpu.sync_copy(col_hbm.at[pl.ds(w * CPW, CPW)], col_v)
    pltpu.sync_copy(ones_hbm, ones_v)
    pltpu.sync_copy(zeros_hbm.at[pl.ds(s * RS, RS)], acc_sh.at[pl.ds(s * RS, RS)])
    plsc.subcore_barrier()

    @pl.loop(0, CPW)
    def _(j):
        pltpu.sync_copy(ones_v, acc_sh.at[col_v.at[j]], add=True)

    plsc.subcore_barrier()
    pltpu.sync_copy(acc_sh.at[pl.ds(s * RS, RS)], out_hbm.at[c, pl.ds(s * RS, RS)])


@functools.partial(
    pl.kernel,
    out_type=[
        jax.ShapeDtypeStruct((4 * NW, CPW, CHUNK), jnp.int32),
        jax.ShapeDtypeStruct((4 * NW, CPW, CHUNK), jnp.int32),
        jax.ShapeDtypeStruct((NW, 16), jnp.int32),
    ],
    mesh=_mesh,
    scratch_types=[
        pltpu.VMEM((CPW, CHUNK), jnp.int32),
        pltpu.VMEM((CPW, CHUNK), jnp.int32),
        pltpu.VMEM((4, CPW, CHUNK), jnp.int32),
        pltpu.VMEM((4, CPW, CHUNK), jnp.int32),
        pltpu.VMEM((16,), jnp.int32),
    ],
    compiler_params=_cp,
)
def _part_kernel(row_hbm, col_hbm, orow_hbm, ocol_hbm, ocnt_hbm,
                 row_v, col_v, orow_b, ocol_b, cnt_s):
    c = lax.axis_index("c")
    s = lax.axis_index("s")
    w = s * 2 + c
    pltpu.sync_copy(row_hbm.at[pl.ds(w * CPW, CPW)], row_v)
    pltpu.sync_copy(col_hbm.at[pl.ds(w * CPW, CPW)], col_v)

    zer16 = jnp.zeros((16,), jnp.int32)
    dump16 = jnp.full((16,), NH, jnp.int32)

    @pl.loop(0, CPW)
    def _(i0):
        for l in range(CHUNK // 16):
            for k in range(4):
                orow_b[k, i0, pl.ds(l * 16, 16)] = zer16
                ocol_b[k, i0, pl.ds(l * 16, 16)] = dump16

    lane = lax.iota(jnp.int32, 16)
    cnt_s[...] = jnp.zeros((16,), jnp.int32)

    @pl.loop(0, CPW)
    def _(j):
        for l in range(CHUNK // 16):
            r16 = row_v[j, pl.ds(l * 16, 16)]
            c16 = col_v[j, pl.ds(l * 16, 16)]
            mr = (r16 >= NH).astype(jnp.int32)
            mc = (c16 >= NH).astype(jnp.int32)
            lr = r16 - mr * NH
            lc = c16 - mc * NH
            kv = mr * 2 + mc
            cv = cnt_s[...]
            inc = jnp.zeros((16,), jnp.int32)
            for k in range(4):
                mk = kv == k
                mki = mk.astype(jnp.int32)
                onek = (lane == k).astype(jnp.int32)
                nk = jnp.sum(mki)
                off = jnp.sum(cv * onek)
                pos = plsc.cumsum(mki) - 1 + off
                plsc.store_scatter(orow_b.at[k], [pos >> 7, pos & 127], lr, mask=mk)
                plsc.store_scatter(ocol_b.at[k], [pos >> 7, pos & 127], lc, mask=mk)
                inc = inc + onek * nk
            cnt_s[...] = cv + inc

    for k in range(4):
        pltpu.sync_copy(orow_b.at[k], orow_hbm.at[k * NW + w])
        pltpu.sync_copy(ocol_b.at[k], ocol_hbm.at[k * NW + w])
    pltpu.sync_copy(cnt_s, ocnt_hbm.at[w])


_PH = 2
_PC = CPW // _PH


@functools.partial(
    pl.kernel,
    out_type=jax.ShapeDtypeStruct((4, NHD, D), jnp.float32),
    mesh=_mesh,
    scratch_types=[
        pltpu.VMEM((_PC, CHUNK), jnp.int32),
        pltpu.VMEM((_PC, CHUNK), jnp.int32),
        pltpu.VMEM((CHUNK, D), jnp.float32),
        pltpu.VMEM((CHUNK, D), jnp.float32),
        pltpu.VMEM((16,), jnp.int32),
        pltpu.VMEM_SHARED((NH, D), jnp.float32),
        pltpu.VMEM_SHARED((NHD, D), jnp.float32),
        pltpu.SemaphoreType.DMA,
        pltpu.SemaphoreType.DMA,
    ],
    compiler_params=_cp,
)
def _agg_kernel(hs_hbm, orow_hbm, ocol_hbm, ocnt_hbm, zeros_hbm, out_hbm,
                row_v, col_v, buf0, buf1, cnt_v, hs_sh, acc_sh, sem0, sem1):
    c = lax.axis_index("c")
    s = lax.axis_index("s")
    hrs = NH // 16
    ars = NHD // 16
    pltpu.sync_copy(hs_hbm.at[pl.ds(c * NH + s * hrs, hrs)],
                    hs_sh.at[pl.ds(s * hrs, hrs)])
    bufs = (buf0, buf1)
    sems = (sem0, sem1)

    for q in range(2):
        k = c * 2 + q
        pltpu.sync_copy(zeros_hbm.at[pl.ds(s * ars, ars)],
                        acc_sh.at[pl.ds(s * ars, ars)])
        plsc.subcore_barrier()
        for slab in range(2):
            slabw = s * 2 + slab
            pltpu.sync_copy(ocnt_hbm.at[slabw], cnt_v)
            lane = lax.iota(jnp.int32, 16)
            cnt = jnp.sum(cnt_v[...] * (lane == k).astype(jnp.int32))
            nchunks = (cnt + CHUNK - 1) >> 7
            for p in range(_PH):
                pltpu.sync_copy(
                    orow_hbm.at[k * NW + slabw, pl.ds(p * _PC, _PC)], row_v)
                pltpu.sync_copy(
                    ocol_hbm.at[k * NW + slabw, pl.ds(p * _PC, _PC)], col_v)
                hi = jnp.clip(nchunks - p * _PC, 0, _PC)

                @pl.loop(0, hi)
                def _(j):
                    pltpu.async_copy(
                        hs_sh.at[row_v.at[j]], bufs[0], sems[0]).wait()
                    pltpu.sync_copy(
                        bufs[0], acc_sh.at[col_v.at[j]], add=True)

        plsc.subcore_barrier()
        pltpu.sync_copy(acc_sh.at[pl.ds(s * ars, ars)],
                        out_hbm.at[c * 2 + q, pl.ds(s * ars, ars)])
        plsc.subcore_barrier()


_BLK = 1024
_GRID = NP // _BLK
_HB = NH // _BLK

_acc_spec_a = pl.BlockSpec((1, _BLK, D), lambda i: (i // _HB, i % _HB, 0))
_acc_spec_b = pl.BlockSpec((1, _BLK, D), lambda i: (2 + i // _HB, i % _HB, 0))


def _mm1_body(x_ref, w_ref, h_ref):
    h_ref[...] = jnp.dot(x_ref[...], w_ref[...],
                         preferred_element_type=jnp.float32,
                         precision=lax.Precision.HIGHEST)


_mm1 = pl.pallas_call(
    _mm1_body,
    grid=(_GRID,),
    in_specs=[
        pl.BlockSpec((_BLK, D), lambda i: (i, 0)),
        pl.BlockSpec((D, D), lambda i: (0, 0)),
    ],
    out_specs=pl.BlockSpec((_BLK, D), lambda i: (i, 0)),
    out_shape=jax.ShapeDtypeStruct((NP, D), jnp.float32),
)


def _scale_body(h_ref, ha_ref, hb_ref, hs_ref, dis_ref):
    dfull = lax.rsqrt(ha_ref[...] + hb_ref[...] + 1.0)
    d = dfull[:, 0:1]
    hs_ref[...] = h_ref[...] * d
    dis_ref[...] = dfull[:, :16]


_scale = pl.pallas_call(
    _scale_body,
    grid=(_GRID,),
    in_specs=[
        pl.BlockSpec((_BLK, D), lambda i: (i, 0)),
        pl.BlockSpec((_BLK, D), lambda i: (i, 0)),
        pl.BlockSpec((_BLK, D), lambda i: (i, 0)),
    ],
    out_specs=[
        pl.BlockSpec((_BLK, D), lambda i: (i, 0)),
        pl.BlockSpec((_BLK, 16), lambda i: (i, 0)),
    ],
    out_shape=[
        jax.ShapeDtypeStruct((NP, D), jnp.float32),
        jax.ShapeDtypeStruct((NP, 16), jnp.float32),
    ],
)


def _combine_mm_body(aa_ref, ab_ref, dis_ref, h1_ref, b_ref, w_ref,
                     h2_ref, hs2_ref):
    d = dis_ref[...][:, 0:1]
    acc = aa_ref[0] + ab_ref[0]
    z = d * acc + (d * d) * h1_ref[...] + b_ref[...]
    r = jnp.maximum(z, 0.0)
    h2 = jnp.dot(r, w_ref[...], preferred_element_type=jnp.float32,
                 precision=lax.Precision.HIGHEST)
    h2_ref[...] = h2
    hs2_ref[...] = h2 * d


_combine_mm = pl.pallas_call(
    _combine_mm_body,
    grid=(_GRID,),
    in_specs=[
        _acc_spec_a,
        _acc_spec_b,
        pl.BlockSpec((_BLK, 16), lambda i: (i, 0)),
        pl.BlockSpec((_BLK, D), lambda i: (i, 0)),
        pl.BlockSpec((1, D), lambda i: (0, 0)),
        pl.BlockSpec((D, D), lambda i: (0, 0)),
    ],
    out_specs=[
        pl.BlockSpec((_BLK, D), lambda i: (i, 0)),
        pl.BlockSpec((_BLK, D), lambda i: (i, 0)),
    ],
    out_shape=[
        jax.ShapeDtypeStruct((NP, D), jnp.float32),
        jax.ShapeDtypeStruct((NP, D), jnp.float32),
    ],
)


def _final_body(aa_ref, ab_ref, dis_ref, h2_ref, b_ref, out_ref):
    d = dis_ref[...][:, 0:1]
    acc = aa_ref[0] + ab_ref[0]
    out_ref[...] = d * acc + (d * d) * h2_ref[...] + b_ref[...]


_final = pl.pallas_call(
    _final_body,
    grid=(_GRID,),
    in_specs=[
        _acc_spec_a,
        _acc_spec_b,
        pl.BlockSpec((_BLK, 16), lambda i: (i, 0)),
        pl.BlockSpec((_BLK, D), lambda i: (i, 0)),
        pl.BlockSpec((1, D), lambda i: (0, 0)),
    ],
    out_specs=pl.BlockSpec((_BLK, D), lambda i: (i, 0)),
    out_shape=jax.ShapeDtypeStruct((NP, D), jnp.float32),
)


def kernel(x, edge_index, W1, b1, W2, b2):
    row = edge_index[0]
    col = edge_index[1]
    col_p = jnp.concatenate(
        [col, jnp.full((EP - E,), N, jnp.int32)]).reshape(EP // CHUNK, CHUNK)
    row_q = jnp.concatenate(
        [row, jnp.zeros((EP - E,), jnp.int32)]).reshape(EP // CHUNK, CHUNK)
    col_q = jnp.concatenate(
        [col, jnp.full((EP - E,), 2 * NH, jnp.int32)]).reshape(EP // CHUNK, CHUNK)
    x_p = jnp.pad(x, ((0, NP - N), (0, 0)))
    zeros128 = jnp.zeros((NP, D), jnp.float32)
    ones128 = jnp.ones((CHUNK, D), jnp.float32)
    b1r = b1.reshape(1, D)
    b2r = b2.reshape(1, D)

    orow, ocol, ocnt = _part_kernel(row_q, col_q)
    h1 = _mm1(x_p, W1)
    hist = _hist_kernel(col_p, ones128, zeros128)
    hs1, dis16 = _scale(h1, hist[0], hist[1])
    acc1 = _agg_kernel(hs1, orow, ocol, ocnt, zeros128)
    h2, hs2 = _combine_mm(acc1, acc1, dis16, h1, b1r, W2)
    acc2 = _agg_kernel(hs2, orow, ocol, ocnt, zeros128)
    out = _final(acc2, acc2, dis16, h2, b2r)
    return out[:N]


# speedup vs baseline: 1.2364x; 1.2364x over previous
import dataclasses
import functools

import jax
import jax.numpy as jnp
from jax import lax
from jax.experimental import pallas as pl
from jax.experimental.pallas import tpu as pltpu
from jax.experimental.pallas import tpu_sc as plsc

N = 10000
D = 128
E = 320000

NP = 10240
NH = NP // 2
NHD = 5248
CHUNK = 128
NW = 32
CPW = 80
EP = NW * CPW * CHUNK
RS = NP // 16
CAP = CPW * CHUNK

_mesh = plsc.VectorSubcoreMesh(core_axis_name="c", subcore_axis_name="s")

_cp = pltpu.CompilerParams()
if "needs_layout_passes" in pltpu.CompilerParams.__dataclass_fields__:
    _cp = dataclasses.replace(_cp, needs_layout_passes=False)


@functools.partial(
    pl.kernel,
    out_type=jax.ShapeDtypeStruct((2, NP, D), jnp.float32),
    mesh=_mesh,
    scratch_types=[
        pltpu.VMEM((CPW, CHUNK), jnp.int32),
        pltpu.VMEM((CHUNK, D), jnp.float32),
        pltpu.VMEM_SHARED((NP, D), jnp.float32),
    ],
)
def _hist_kernel(col_hbm, ones_hbm, zeros_hbm, out_hbm, col_v, ones_v, acc_sh):
    c = lax.axis_index("c")
    s = lax.axis_index("s")
    w = s * 2 + c
    pltpu.sync_copy(col_hbm.at[pl.ds(w * CPW, CPW)], col_v)
    pltpu.sync_copy(ones_hbm, ones_v)
    pltpu.sync_copy(zeros_hbm.at[pl.ds(s * RS, RS)], acc_sh.at[pl.ds(s * RS, RS)])
    plsc.subcore_barrier()

    @pl.loop(0, CPW)
    def _(j):
        pltpu.sync_copy(ones_v, acc_sh.at[col_v.at[j]], add=True)

    plsc.subcore_barrier()
    pltpu.sync_copy(acc_sh.at[pl.ds(s * RS, RS)], out_hbm.at[c, pl.ds(s * RS, RS)])


@functools.partial(
    pl.kernel,
    out_type=[
        jax.ShapeDtypeStruct((4 * NW, CPW, CHUNK), jnp.int32),
        jax.ShapeDtypeStruct((4 * NW, CPW, CHUNK), jnp.int32),
        jax.ShapeDtypeStruct((NW, 16), jnp.int32),
    ],
    mesh=_mesh,
    scratch_types=[
        pltpu.VMEM((CPW, CHUNK), jnp.int32),
        pltpu.VMEM((CPW, CHUNK), jnp.int32),
        pltpu.VMEM((4, CPW, CHUNK), jnp.int32),
        pltpu.VMEM((4, CPW, CHUNK), jnp.int32),
        pltpu.VMEM((16,), jnp.int32),
    ],
    compiler_params=_cp,
)
def _part_kernel(row_hbm, col_hbm, orow_hbm, ocol_hbm, ocnt_hbm,
                 row_v, col_v, orow_b, ocol_b, cnt_s):
    c = lax.axis_index("c")
    s = lax.axis_index("s")
    w = s * 2 + c
    pltpu.sync_copy(row_hbm.at[pl.ds(w * CPW, CPW)], row_v)
    pltpu.sync_copy(col_hbm.at[pl.ds(w * CPW, CPW)], col_v)

    zer16 = jnp.zeros((16,), jnp.int32)
    dump16 = jnp.full((16,), NH, jnp.int32)

    @pl.loop(0, CPW)
    def _(i0):
        for l in range(CHUNK // 16):
            for k in range(4):
                orow_b[k, i0, pl.ds(l * 16, 16)] = zer16
                ocol_b[k, i0, pl.ds(l * 16, 16)] = dump16

    lane = lax.iota(jnp.int32, 16)
    cnt_s[...] = jnp.zeros((16,), jnp.int32)

    @pl.loop(0, CPW)
    def _(j):
        for l in range(CHUNK // 16):
            r16 = row_v[j, pl.ds(l * 16, 16)]
            c16 = col_v[j, pl.ds(l * 16, 16)]
            mr = (r16 >= NH).astype(jnp.int32)
            mc = (c16 >= NH).astype(jnp.int32)
            lr = r16 - mr * NH
            lc = c16 - mc * NH
            kv = mr * 2 + mc
            cv = cnt_s[...]
            inc = jnp.zeros((16,), jnp.int32)
            for k in range(4):
                mk = kv == k
                mki = mk.astype(jnp.int32)
                onek = (lane == k).astype(jnp.int32)
                nk = jnp.sum(mki)
                off = jnp.sum(cv * onek)
                pos = plsc.cumsum(mki) - 1 + off
                plsc.store_scatter(orow_b.at[k], [pos >> 7, pos & 127], lr, mask=mk)
                plsc.store_scatter(ocol_b.at[k], [pos >> 7, pos & 127], lc, mask=mk)
                inc = inc + onek * nk
            cnt_s[...] = cv + inc

    for k in range(4):
        pltpu.sync_copy(orow_b.at[k], orow_hbm.at[k * NW + w])
        pltpu.sync_copy(ocol_b.at[k], ocol_hbm.at[k * NW + w])
    pltpu.sync_copy(cnt_s, ocnt_hbm.at[w])


_PH = 2
_PC = CPW // _PH


@functools.partial(
    pl.kernel,
    out_type=jax.ShapeDtypeStruct((4, NHD, D), jnp.float32),
    mesh=_mesh,
    scratch_types=[
        pltpu.VMEM((_PC, CHUNK), jnp.int32),
        pltpu.VMEM((_PC, CHUNK), jnp.int32),
        pltpu.VMEM((CHUNK, D), jnp.float32),
        pltpu.VMEM((CHUNK, D), jnp.float32),
        pltpu.VMEM((16,), jnp.int32),
        pltpu.VMEM_SHARED((NH, D), jnp.float32),
        pltpu.VMEM_SHARED((NHD, D), jnp.float32),
        pltpu.SemaphoreType.DMA,
        pltpu.SemaphoreType.DMA,
    ],
    compiler_params=_cp,
)
def _agg_kernel(hs_hbm, orow_hbm, ocol_hbm, ocnt_hbm, zeros_hbm, out_hbm,
                row_v, col_v, buf0, buf1, cnt_v, hs_sh, acc_sh, sem0, sem1):
    c = lax.axis_index("c")
    s = lax.axis_index("s")
    hrs = NH // 16
    ars = NHD // 16
    pltpu.sync_copy(hs_hbm.at[pl.ds(c * NH + s * hrs, hrs)],
                    hs_sh.at[pl.ds(s * hrs, hrs)])
    bufs = (buf0, buf1)
    sems = (sem0, sem1)

    for q in range(2):
        k = c * 2 + q
        pltpu.sync_copy(zeros_hbm.at[pl.ds(s * ars, ars)],
                        acc_sh.at[pl.ds(s * ars, ars)])
        plsc.subcore_barrier()
        for slab in range(2):
            slabw = s * 2 + slab
            pltpu.sync_copy(ocnt_hbm.at[slabw], cnt_v)
            lane = lax.iota(jnp.int32, 16)
            cnt = jnp.sum(cnt_v[...] * (lane == k).astype(jnp.int32))
            nchunks = (cnt + CHUNK - 1) >> 7
            for p in range(_PH):
                pltpu.sync_copy(
                    orow_hbm.at[k * NW + slabw, pl.ds(p * _PC, _PC)], row_v)
                pltpu.sync_copy(
                    ocol_hbm.at[k * NW + slabw, pl.ds(p * _PC, _PC)], col_v)
                hi = jnp.clip(nchunks - p * _PC, 0, _PC)

                @pl.when(hi > 0)
                def _():
                    pltpu.make_async_copy(
                        hs_sh.at[row_v.at[0]], bufs[0], sems[0]).start()

                    @pl.loop(0, hi)
                    def _(j):
                        for b in range(2):

                            @pl.when((j & 1) == b)
                            def _():
                                pltpu.make_async_copy(
                                    hs_sh.at[row_v.at[j]], bufs[b],
                                    sems[b]).wait()

                                @pl.when(j + 1 < hi)
                                def _():
                                    pltpu.make_async_copy(
                                        hs_sh.at[row_v.at[j + 1]],
                                        bufs[1 - b], sems[1 - b]).start()

                                pltpu.sync_copy(
                                    bufs[b], acc_sh.at[col_v.at[j]], add=True)

        plsc.subcore_barrier()
        pltpu.sync_copy(acc_sh.at[pl.ds(s * ars, ars)],
                        out_hbm.at[c * 2 + q, pl.ds(s * ars, ars)])
        plsc.subcore_barrier()


_BLK = 1024
_GRID = NP // _BLK
_HB = NH // _BLK

_acc_spec_a = pl.BlockSpec((1, _BLK, D), lambda i: (i // _HB, i % _HB, 0))
_acc_spec_b = pl.BlockSpec((1, _BLK, D), lambda i: (2 + i // _HB, i % _HB, 0))


def _mm1_body(x_ref, w_ref, h_ref):
    h_ref[...] = jnp.dot(x_ref[...], w_ref[...],
                         preferred_element_type=jnp.float32,
                         precision=lax.Precision.HIGHEST)


_mm1 = pl.pallas_call(
    _mm1_body,
    grid=(_GRID,),
    in_specs=[
        pl.BlockSpec((_BLK, D), lambda i: (i, 0)),
        pl.BlockSpec((D, D), lambda i: (0, 0)),
    ],
    out_specs=pl.BlockSpec((_BLK, D), lambda i: (i, 0)),
    out_shape=jax.ShapeDtypeStruct((NP, D), jnp.float32),
)


def _scale_body(h_ref, ha_ref, hb_ref, hs_ref, dis_ref):
    dfull = lax.rsqrt(ha_ref[...] + hb_ref[...] + 1.0)
    d = dfull[:, 0:1]
    hs_ref[...] = h_ref[...] * d
    dis_ref[...] = dfull[:, :16]


_scale = pl.pallas_call(
    _scale_body,
    grid=(_GRID,),
    in_specs=[
        pl.BlockSpec((_BLK, D), lambda i: (i, 0)),
        pl.BlockSpec((_BLK, D), lambda i: (i, 0)),
        pl.BlockSpec((_BLK, D), lambda i: (i, 0)),
    ],
    out_specs=[
        pl.BlockSpec((_BLK, D), lambda i: (i, 0)),
        pl.BlockSpec((_BLK, 16), lambda i: (i, 0)),
    ],
    out_shape=[
        jax.ShapeDtypeStruct((NP, D), jnp.float32),
        jax.ShapeDtypeStruct((NP, 16), jnp.float32),
    ],
)


def _combine_mm_body(aa_ref, ab_ref, dis_ref, h1_ref, b_ref, w_ref,
                     h2_ref, hs2_ref):
    d = dis_ref[...][:, 0:1]
    acc = aa_ref[0] + ab_ref[0]
    z = d * acc + (d * d) * h1_ref[...] + b_ref[...]
    r = jnp.maximum(z, 0.0)
    h2 = jnp.dot(r, w_ref[...], preferred_element_type=jnp.float32,
                 precision=lax.Precision.HIGHEST)
    h2_ref[...] = h2
    hs2_ref[...] = h2 * d


_combine_mm = pl.pallas_call(
    _combine_mm_body,
    grid=(_GRID,),
    in_specs=[
        _acc_spec_a,
        _acc_spec_b,
        pl.BlockSpec((_BLK, 16), lambda i: (i, 0)),
        pl.BlockSpec((_BLK, D), lambda i: (i, 0)),
        pl.BlockSpec((1, D), lambda i: (0, 0)),
        pl.BlockSpec((D, D), lambda i: (0, 0)),
    ],
    out_specs=[
        pl.BlockSpec((_BLK, D), lambda i: (i, 0)),
        pl.BlockSpec((_BLK, D), lambda i: (i, 0)),
    ],
    out_shape=[
        jax.ShapeDtypeStruct((NP, D), jnp.float32),
        jax.ShapeDtypeStruct((NP, D), jnp.float32),
    ],
)


def _final_body(aa_ref, ab_ref, dis_ref, h2_ref, b_ref, out_ref):
    d = dis_ref[...][:, 0:1]
    acc = aa_ref[0] + ab_ref[0]
    out_ref[...] = d * acc + (d * d) * h2_ref[...] + b_ref[...]


_final = pl.pallas_call(
    _final_body,
    grid=(_GRID,),
    in_specs=[
        _acc_spec_a,
        _acc_spec_b,
        pl.BlockSpec((_BLK, 16), lambda i: (i, 0)),
        pl.BlockSpec((_BLK, D), lambda i: (i, 0)),
        pl.BlockSpec((1, D), lambda i: (0, 0)),
    ],
    out_specs=pl.BlockSpec((_BLK, D), lambda i: (i, 0)),
    out_shape=jax.ShapeDtypeStruct((NP, D), jnp.float32),
)


def kernel(x, edge_index, W1, b1, W2, b2):
    row = edge_index[0]
    col = edge_index[1]
    col_p = jnp.concatenate(
        [col, jnp.full((EP - E,), N, jnp.int32)]).reshape(EP // CHUNK, CHUNK)
    row_q = jnp.concatenate(
        [row, jnp.zeros((EP - E,), jnp.int32)]).reshape(EP // CHUNK, CHUNK)
    col_q = jnp.concatenate(
        [col, jnp.full((EP - E,), 2 * NH, jnp.int32)]).reshape(EP // CHUNK, CHUNK)
    x_p = jnp.pad(x, ((0, NP - N), (0, 0)))
    zeros128 = jnp.zeros((NP, D), jnp.float32)
    ones128 = jnp.ones((CHUNK, D), jnp.float32)
    b1r = b1.reshape(1, D)
    b2r = b2.reshape(1, D)

    orow, ocol, ocnt = _part_kernel(row_q, col_q)
    h1 = _mm1(x_p, W1)
    hist = _hist_kernel(col_p, ones128, zeros128)
    hs1, dis16 = _scale(h1, hist[0], hist[1])
    acc1 = _agg_kernel(hs1, orow, ocol, ocnt, zeros128)
    h2, hs2 = _combine_mm(acc1, acc1, dis16, h1, b1r, W2)
    acc2 = _agg_kernel(hs2, orow, ocol, ocnt, zeros128)
    out = _final(acc2, acc2, dis16, h2, b2r)
    return out[:N]


# speedup vs baseline: 1.2520x; 1.0126x over previous
import dataclasses
import functools

import jax
import jax.numpy as jnp
from jax import lax
from jax.experimental import pallas as pl
from jax.experimental.pallas import tpu as pltpu
from jax.experimental.pallas import tpu_sc as plsc

N = 10000
D = 128
E = 320000

NP = 10240
NH = NP // 2
NHD = 5248
CHUNK = 128
NW = 32
CPW = 80
EP = NW * CPW * CHUNK
RS = NP // 16
CAP = CPW * CHUNK

_mesh = plsc.VectorSubcoreMesh(core_axis_name="c", subcore_axis_name="s")

_cp = pltpu.CompilerParams()
if "needs_layout_passes" in pltpu.CompilerParams.__dataclass_fields__:
    _cp = dataclasses.replace(_cp, needs_layout_passes=False)


@functools.partial(
    pl.kernel,
    out_type=jax.ShapeDtypeStruct((2, NP, D), jnp.float32),
    mesh=_mesh,
    scratch_types=[
        pltpu.VMEM((CPW, CHUNK), jnp.int32),
        pltpu.VMEM((CHUNK, D), jnp.float32),
        pltpu.VMEM_SHARED((NP, D), jnp.float32),
    ],
)
def _hist_kernel(col_hbm, ones_hbm, zeros_hbm, out_hbm, col_v, ones_v, acc_sh):
    c = lax.axis_index("c")
    s = lax.axis_index("s")
    w = s * 2 + c
    pltpu.sync_copy(col_hbm.at[pl.ds(w * CPW, CPW)], col_v)
    pltpu.sync_copy(ones_hbm, ones_v)
    pltpu.sync_copy(zeros_hbm.at[pl.ds(s * RS, RS)], acc_sh.at[pl.ds(s * RS, RS)])
    plsc.subcore_barrier()

    @pl.loop(0, CPW)
    def _(j):
        pltpu.sync_copy(ones_v, acc_sh.at[col_v.at[j]], add=True)

    plsc.subcore_barrier()
    pltpu.sync_copy(acc_sh.at[pl.ds(s * RS, RS)], out_hbm.at[c, pl.ds(s * RS, RS)])


@functools.partial(
    pl.kernel,
    out_type=[
        jax.ShapeDtypeStruct((4 * NW, CPW, CHUNK), jnp.int32),
        jax.ShapeDtypeStruct((4 * NW, CPW, CHUNK), jnp.int32),
        jax.ShapeDtypeStruct((NW, 16), jnp.int32),
    ],
    mesh=_mesh,
    scratch_types=[
        pltpu.VMEM((CPW, CHUNK), jnp.int32),
        pltpu.VMEM((CPW, CHUNK), jnp.int32),
        pltpu.VMEM((4, CPW, CHUNK), jnp.int32),
        pltpu.VMEM((4, CPW, CHUNK), jnp.int32),
        pltpu.VMEM((16,), jnp.int32),
    ],
    compiler_params=_cp,
)
def _part_kernel(row_hbm, col_hbm, orow_hbm, ocol_hbm, ocnt_hbm,
                 row_v, col_v, orow_b, ocol_b, cnt_s):
    c = lax.axis_index("c")
    s = lax.axis_index("s")
    w = s * 2 + c
    pltpu.sync_copy(row_hbm.at[pl.ds(w * CPW, CPW)], row_v)
    pltpu.sync_copy(col_hbm.at[pl.ds(w * CPW, CPW)], col_v)

    zer16 = jnp.zeros((16,), jnp.int32)
    dump16 = jnp.full((16,), NH, jnp.int32)

    @pl.loop(0, CPW)
    def _(i0):
        for l in range(CHUNK // 16):
            for k in range(4):
                orow_b[k, i0, pl.ds(l * 16, 16)] = zer16
                ocol_b[k, i0, pl.ds(l * 16, 16)] = dump16

    lane = lax.iota(jnp.int32, 16)
    cnt_s[...] = jnp.zeros((16,), jnp.int32)

    @pl.loop(0, CPW)
    def _(j):
        for l in range(CHUNK // 16):
            r16 = row_v[j, pl.ds(l * 16, 16)]
            c16 = col_v[j, pl.ds(l * 16, 16)]
            mr = (r16 >= NH).astype(jnp.int32)
            mc = (c16 >= NH).astype(jnp.int32)
            lr = r16 - mr * NH
            lc = c16 - mc * NH
            kv = mr * 2 + mc
            cv = cnt_s[...]
            inc = jnp.zeros((16,), jnp.int32)
            for k in range(4):
                mk = kv == k
                mki = mk.astype(jnp.int32)
                onek = (lane == k).astype(jnp.int32)
                nk = jnp.sum(mki)
                off = jnp.sum(cv * onek)
                pos = plsc.cumsum(mki) - 1 + off
                plsc.store_scatter(orow_b.at[k], [pos >> 7, pos & 127], lr, mask=mk)
                plsc.store_scatter(ocol_b.at[k], [pos >> 7, pos & 127], lc, mask=mk)
                inc = inc + onek * nk
            cnt_s[...] = cv + inc

    for k in range(4):
        pltpu.sync_copy(orow_b.at[k], orow_hbm.at[k * NW + w])
        pltpu.sync_copy(ocol_b.at[k], ocol_hbm.at[k * NW + w])
    pltpu.sync_copy(cnt_s, ocnt_hbm.at[w])


_PH = 2
_PC = CPW // _PH


@functools.partial(
    pl.kernel,
    out_type=jax.ShapeDtypeStruct((4, NHD, D), jnp.float32),
    mesh=_mesh,
    scratch_types=[
        pltpu.VMEM((_PC, CHUNK), jnp.int32),
        pltpu.VMEM((_PC, CHUNK), jnp.int32),
        pltpu.VMEM((CHUNK, D), jnp.float32),
        pltpu.VMEM((CHUNK, D), jnp.float32),
        pltpu.VMEM((16,), jnp.int32),
        pltpu.VMEM_SHARED((NH, D), jnp.float32),
        pltpu.VMEM_SHARED((NHD, D), jnp.float32),
        pltpu.SemaphoreType.DMA,
        pltpu.SemaphoreType.DMA,
    ],
    compiler_params=_cp,
)
def _agg_kernel(hs_hbm, orow_hbm, ocol_hbm, ocnt_hbm, zeros_hbm, out_hbm,
                row_v, col_v, buf0, buf1, cnt_v, hs_sh, acc_sh, sem0, sem1):
    c = lax.axis_index("c")
    s = lax.axis_index("s")
    hrs = NH // 16
    ars = NHD // 16
    pltpu.sync_copy(hs_hbm.at[pl.ds(c * NH + s * hrs, hrs)],
                    hs_sh.at[pl.ds(s * hrs, hrs)])
    bufs = (buf0, buf1)
    sems = (sem0, sem1)

    for q in range(2):
        k = c * 2 + q
        pltpu.sync_copy(zeros_hbm.at[pl.ds(s * ars, ars)],
                        acc_sh.at[pl.ds(s * ars, ars)])
        plsc.subcore_barrier()
        for slab in range(2):
            slabw = s * 2 + slab
            pltpu.sync_copy(ocnt_hbm.at[slabw], cnt_v)
            lane = lax.iota(jnp.int32, 16)
            cnt = jnp.sum(cnt_v[...] * (lane == k).astype(jnp.int32))
            nchunks = (cnt + CHUNK - 1) >> 7
            for p in range(_PH):
                hi = jnp.clip(nchunks - p * _PC, 0, _PC)

                @pl.when(hi > 0)
                def _():
                    pltpu.sync_copy(
                        orow_hbm.at[k * NW + slabw, pl.ds(p * _PC, _PC)], row_v)
                    pltpu.sync_copy(
                        ocol_hbm.at[k * NW + slabw, pl.ds(p * _PC, _PC)], col_v)
                    pltpu.make_async_copy(
                        hs_sh.at[row_v.at[0]], bufs[0], sems[0]).start()

                    @pl.loop(0, hi)
                    def _(j):
                        for b in range(2):

                            @pl.when((j & 1) == b)
                            def _():
                                pltpu.make_async_copy(
                                    hs_sh.at[row_v.at[j]], bufs[b],
                                    sems[b]).wait()

                                @pl.when(j + 1 < hi)
                                def _():
                                    pltpu.make_async_copy(
                                        hs_sh.at[row_v.at[j + 1]],
                                        bufs[1 - b], sems[1 - b]).start()

                                pltpu.sync_copy(
                                    bufs[b], acc_sh.at[col_v.at[j]], add=True)

        plsc.subcore_barrier()
        pltpu.sync_copy(acc_sh.at[pl.ds(s * ars, ars)],
                        out_hbm.at[c * 2 + q, pl.ds(s * ars, ars)])
        plsc.subcore_barrier()


_BLK = 1024
_GRID = NP // _BLK
_HB = NH // _BLK

_acc_spec_a = pl.BlockSpec((1, _BLK, D), lambda i: (i // _HB, i % _HB, 0))
_acc_spec_b = pl.BlockSpec((1, _BLK, D), lambda i: (2 + i // _HB, i % _HB, 0))


def _mm1_body(x_ref, w_ref, h_ref):
    h_ref[...] = jnp.dot(x_ref[...], w_ref[...],
                         preferred_element_type=jnp.float32,
                         precision=lax.Precision.HIGHEST)


_mm1 = pl.pallas_call(
    _mm1_body,
    grid=(_GRID,),
    in_specs=[
        pl.BlockSpec((_BLK, D), lambda i: (i, 0)),
        pl.BlockSpec((D, D), lambda i: (0, 0)),
    ],
    out_specs=pl.BlockSpec((_BLK, D), lambda i: (i, 0)),
    out_shape=jax.ShapeDtypeStruct((NP, D), jnp.float32),
)


def _scale_body(h_ref, ha_ref, hb_ref, hs_ref, dis_ref):
    dfull = lax.rsqrt(ha_ref[...] + hb_ref[...] + 1.0)
    d = dfull[:, 0:1]
    hs_ref[...] = h_ref[...] * d
    dis_ref[...] = dfull[:, :16]


_scale = pl.pallas_call(
    _scale_body,
    grid=(_GRID,),
    in_specs=[
        pl.BlockSpec((_BLK, D), lambda i: (i, 0)),
        pl.BlockSpec((_BLK, D), lambda i: (i, 0)),
        pl.BlockSpec((_BLK, D), lambda i: (i, 0)),
    ],
    out_specs=[
        pl.BlockSpec((_BLK, D), lambda i: (i, 0)),
        pl.BlockSpec((_BLK, 16), lambda i: (i, 0)),
    ],
    out_shape=[
        jax.ShapeDtypeStruct((NP, D), jnp.float32),
        jax.ShapeDtypeStruct((NP, 16), jnp.float32),
    ],
)


def _combine_mm_body(aa_ref, ab_ref, dis_ref, h1_ref, b_ref, w_ref,
                     h2_ref, hs2_ref):
    d = dis_ref[...][:, 0:1]
    acc = aa_ref[0] + ab_ref[0]
    z = d * acc + (d * d) * h1_ref[...] + b_ref[...]
    r = jnp.maximum(z, 0.0)
    h2 = jnp.dot(r, w_ref[...], preferred_element_type=jnp.float32,
                 precision=lax.Precision.HIGHEST)
    h2_ref[...] = h2
    hs2_ref[...] = h2 * d


_combine_mm = pl.pallas_call(
    _combine_mm_body,
    grid=(_GRID,),
    in_specs=[
        _acc_spec_a,
        _acc_spec_b,
        pl.BlockSpec((_BLK, 16), lambda i: (i, 0)),
        pl.BlockSpec((_BLK, D), lambda i: (i, 0)),
        pl.BlockSpec((1, D), lambda i: (0, 0)),
        pl.BlockSpec((D, D), lambda i: (0, 0)),
    ],
    out_specs=[
        pl.BlockSpec((_BLK, D), lambda i: (i, 0)),
        pl.BlockSpec((_BLK, D), lambda i: (i, 0)),
    ],
    out_shape=[
        jax.ShapeDtypeStruct((NP, D), jnp.float32),
        jax.ShapeDtypeStruct((NP, D), jnp.float32),
    ],
)


def _final_body(aa_ref, ab_ref, dis_ref, h2_ref, b_ref, out_ref):
    d = dis_ref[...][:, 0:1]
    acc = aa_ref[0] + ab_ref[0]
    out_ref[...] = d * acc + (d * d) * h2_ref[...] + b_ref[...]


_final = pl.pallas_call(
    _final_body,
    grid=(_GRID,),
    in_specs=[
        _acc_spec_a,
        _acc_spec_b,
        pl.BlockSpec((_BLK, 16), lambda i: (i, 0)),
        pl.BlockSpec((_BLK, D), lambda i: (i, 0)),
        pl.BlockSpec((1, D), lambda i: (0, 0)),
    ],
    out_specs=pl.BlockSpec((_BLK, D), lambda i: (i, 0)),
    out_shape=jax.ShapeDtypeStruct((NP, D), jnp.float32),
)


def kernel(x, edge_index, W1, b1, W2, b2):
    row = edge_index[0]
    col = edge_index[1]
    col_p = jnp.concatenate(
        [col, jnp.full((EP - E,), N, jnp.int32)]).reshape(EP // CHUNK, CHUNK)
    row_q = jnp.concatenate(
        [row, jnp.zeros((EP - E,), jnp.int32)]).reshape(EP // CHUNK, CHUNK)
    col_q = jnp.concatenate(
        [col, jnp.full((EP - E,), 2 * NH, jnp.int32)]).reshape(EP // CHUNK, CHUNK)
    x_p = jnp.pad(x, ((0, NP - N), (0, 0)))
    zeros128 = jnp.zeros((NP, D), jnp.float32)
    ones128 = jnp.ones((CHUNK, D), jnp.float32)
    b1r = b1.reshape(1, D)
    b2r = b2.reshape(1, D)

    orow, ocol, ocnt = _part_kernel(row_q, col_q)
    h1 = _mm1(x_p, W1)
    hist = _hist_kernel(col_p, ones128, zeros128)
    hs1, dis16 = _scale(h1, hist[0], hist[1])
    acc1 = _agg_kernel(hs1, orow, ocol, ocnt, zeros128)
    h2, hs2 = _combine_mm(acc1, acc1, dis16, h1, b1r, W2)
    acc2 = _agg_kernel(hs2, orow, ocol, ocnt, zeros128)
    out = _final(acc2, acc2, dis16, h2, b2r)
    return out[:N]
